# baseline (speedup 1.0000x reference)
import functools

import jax
import jax.numpy as jnp
from jax import lax
from jax.experimental import pallas as pl
from jax.experimental.pallas import tpu as pltpu
from jax.experimental.pallas import tpu_sc as plsc

_NVARS = 99
_SHAPE = (8, 40320, _NVARS)
_R = _SHAPE[0]
_M = _SHAPE[1]
_NW = 32
_CHUNK_L = 21 * 128
_MCH = _M // _CHUNK_L
_ITEMS = _NVARS * _MCH
_NTRIP = 16
_NVEC = _CHUNK_L // 16


def _sc_norm(xt):
    mesh = plsc.VectorSubcoreMesh(core_axis_name="c", subcore_axis_name="s")

    @functools.partial(
        pl.kernel,
        mesh=mesh,
        out_type=jax.ShapeDtypeStruct((_NVARS, _R, _M), jnp.float32),
        scratch_types=[
            pltpu.VMEM((_R, _CHUNK_L), jnp.float32),
            pltpu.VMEM((_R, _CHUNK_L), jnp.float32),
            pltpu.VMEM((_R, _CHUNK_L), jnp.float32),
            pltpu.SemaphoreType.DMA,
            pltpu.SemaphoreType.DMA,
            pltpu.SemaphoreType.DMA,
            pltpu.SemaphoreType.DMA,
            pltpu.SemaphoreType.DMA,
            pltpu.SemaphoreType.DMA,
        ],
        compiler_params=pltpu.CompilerParams(use_tc_tiling_on_sc=True),
    )
    def k(x_hbm, out_hbm, b0, b1, b2, si0, si1, si2, so0, so1, so2):
        w = lax.axis_index("s") * 2 + lax.axis_index("c")
        bufs = (b0, b1, b2)
        sis = (si0, si1, si2)
        sos = (so0, so1, so2)

        def item(s):
            return s * _NW + w

        def valid(s):
            return item(s) < _ITEMS

        def in_sl(s):
            i = item(s)
            return x_hbm.at[i // _MCH, :, pl.ds((i % _MCH) * _CHUNK_L, _CHUNK_L)]

        def out_sl(s):
            i = item(s)
            return out_hbm.at[i // _MCH, :, pl.ds((i % _MCH) * _CHUNK_L, _CHUNK_L)]

        def coeffs(s):
            ci = item(s) // _MCH
            cf = ci.astype(jnp.float32)
            denom = jnp.where(ci < 10, 1.0,
                              jnp.where(ci < 13, cf + 1.0, 0.01 * cf + 1.0))
            mufac = jnp.where(ci < 13, 0.0, 0.1 * cf)
            dv = lax.broadcast(denom, (16,))
            av = jnp.ones((16,), jnp.float32) / dv
            bv = -lax.broadcast(mufac, (16,)) * av
            return av, bv

        def compute(buf, s):
            av, bv = coeffs(s)

            @plsc.parallel_loop(0, _R, step=1, unroll=4)
            def _(r):
                for j in range(_NVEC):
                    sl = pl.ds(j * 16, 16)
                    buf[r, sl] = buf[r, sl] * av + bv

        pltpu.async_copy(in_sl(0), b0, si0)
        pltpu.async_copy(in_sl(1), b1, si1)

        def triple(t, carry):
            for b in range(3):
                s = t * 3 + b
                buf, si, so = bufs[b], sis[b], sos[b]
                pb = (b - 1) % 3
                pbuf, psi, pso = bufs[pb], sis[pb], sos[pb]

                @pl.when(valid(s))
                def _():
                    pltpu.make_async_copy(in_sl(s), buf, si).wait()

                if b == 0:
                    prev_ok = jnp.logical_and(t >= 1, valid(s - 1))
                else:
                    prev_ok = valid(s - 1)

                @pl.when(prev_ok)
                def _():
                    pltpu.make_async_copy(pbuf, out_sl(s - 1), pso).wait()

                @pl.when(valid(s + 2))
                def _():
                    pltpu.async_copy(in_sl(s + 2), pbuf, psi)

                @pl.when(valid(s))
                def _():
                    compute(buf, s)
                    pltpu.async_copy(buf, out_sl(s), so)
            return carry

        lax.fori_loop(0, _NTRIP, triple, 0)

    return k(xt)


@functools.partial(jax.jit)
def kernel(x):
    xt = jnp.transpose(x, (2, 0, 1))
    out_t = _sc_norm(xt)
    return jnp.transpose(out_t, (1, 2, 0))


# speedup vs baseline: 1.0633x; 1.0633x over previous
import functools

import jax
import jax.numpy as jnp
from jax import lax
from jax.experimental import pallas as pl
from jax.experimental.pallas import tpu as pltpu
from jax.experimental.pallas import tpu_sc as plsc

_NVARS = 99
_SHAPE = (8, 40320, _NVARS)
_R = _SHAPE[0]
_M = _SHAPE[1]
_NW = 32
_CHUNK_L = 21 * 128
_MCH = _M // _CHUNK_L
_ITEMS = _NVARS * _MCH
_NTRIP = 16
_NVEC = _CHUNK_L // 16


def _sc_norm(xt):
    mesh = plsc.VectorSubcoreMesh(core_axis_name="c", subcore_axis_name="s")

    @functools.partial(
        pl.kernel,
        mesh=mesh,
        out_type=jax.ShapeDtypeStruct((_NVARS, _R, _M), jnp.float32),
        scratch_types=[
            pltpu.VMEM((_R, _CHUNK_L), jnp.float32),
            pltpu.VMEM((_R, _CHUNK_L), jnp.float32),
            pltpu.VMEM((_R, _CHUNK_L), jnp.float32),
            pltpu.SemaphoreType.DMA,
            pltpu.SemaphoreType.DMA,
            pltpu.SemaphoreType.DMA,
            pltpu.SemaphoreType.DMA,
            pltpu.SemaphoreType.DMA,
            pltpu.SemaphoreType.DMA,
        ],
        compiler_params=pltpu.CompilerParams(use_tc_tiling_on_sc=True),
    )
    def k(x_hbm, out_hbm, b0, b1, b2, si0, si1, si2, so0, so1, so2):
        w = lax.axis_index("s") * 2 + lax.axis_index("c")
        bufs = (b0, b1, b2)
        sis = (si0, si1, si2)
        sos = (so0, so1, so2)

        def item(s):
            return s * _NW + w

        def valid(s):
            return item(s) < _ITEMS

        def in_sl(s):
            i = item(s)
            return x_hbm.at[i // _MCH, :, pl.ds((i % _MCH) * _CHUNK_L, _CHUNK_L)]

        def out_sl(s):
            i = item(s)
            return out_hbm.at[i // _MCH, :, pl.ds((i % _MCH) * _CHUNK_L, _CHUNK_L)]

        def coeffs(s):
            ci = item(s) // _MCH
            cf = ci.astype(jnp.float32)
            denom = jnp.where(ci < 10, 1.0,
                              jnp.where(ci < 13, cf + 1.0, 0.01 * cf + 1.0))
            mufac = jnp.where(ci < 13, 0.0, 0.1 * cf)
            dv = lax.broadcast(denom, (16,))
            av = jnp.ones((16,), jnp.float32) / dv
            bv = -lax.broadcast(mufac, (16,)) * av
            return av, bv

        def compute(buf, s):
            av, bv = coeffs(s)

            @plsc.parallel_loop(0, _R, step=1, unroll=2)
            def _(r):
                for j in range(_NVEC):
                    sl = pl.ds(j * 16, 16)
                    buf[r, sl] = buf[r, sl] * av + bv

        pltpu.async_copy(in_sl(0), b0, si0)
        pltpu.async_copy(in_sl(1), b1, si1)

        def triple(t, carry):
            for b in range(3):
                s = t * 3 + b
                buf, si, so = bufs[b], sis[b], sos[b]
                pb = (b - 1) % 3
                pbuf, psi, pso = bufs[pb], sis[pb], sos[pb]

                @pl.when(valid(s))
                def _():
                    pltpu.make_async_copy(in_sl(s), buf, si).wait()

                if b == 0:
                    prev_ok = jnp.logical_and(t >= 1, valid(s - 1))
                else:
                    prev_ok = valid(s - 1)

                @pl.when(prev_ok)
                def _():
                    pltpu.make_async_copy(pbuf, out_sl(s - 1), pso).wait()

                @pl.when(valid(s + 2))
                def _():
                    pltpu.async_copy(in_sl(s + 2), pbuf, psi)

                @pl.when(valid(s))
                def _():
                    compute(buf, s)
                    pltpu.async_copy(buf, out_sl(s), so)
            return carry

        lax.fori_loop(0, _NTRIP, triple, 0)

    return k(xt)


@functools.partial(jax.jit)
def kernel(x):
    xt = jnp.transpose(x, (2, 0, 1))
    out_t = _sc_norm(xt)
    return jnp.transpose(out_t, (1, 2, 0))
